# baseline (speedup 1.0000x reference)
import jax
import jax.numpy as jnp
from jax import lax
from jax.experimental import pallas as pl
from jax.experimental.pallas import tpu as pltpu
from jax.experimental.pallas import tpu_sc as plsc

B = 4096
H = 50
D = 32
L = 16
NC = 2
NS = 16
NW = NC * NS
BPW = B // NW
RPD = 2
IPD = RPD * H
NDMA = BPW // RPD
INV_H = 1.0 / H


def _body(ids_hbm, table_hbm, out_hbm, idx_v, buf0, buf1, out_v, sem0, sem1):
    wid = lax.axis_index("s") * NC + lax.axis_index("c")

    pltpu.sync_copy(ids_hbm.at[wid], idx_v)

    def start(j, buf, sem):
        pltpu.async_copy(table_hbm.at[idx_v.at[j]], buf, sem)

    def wait(j, buf, sem):
        pltpu.make_async_copy(table_hbm.at[idx_v.at[j]], buf, sem).wait()

    def reduce_pair(j, buf):
        for h in range(RPD):
            def add_row(t, carry, h=h):
                a0, a1 = carry
                r = h * H + t
                return (a0 + buf[r, pl.ds(0, L)], a1 + buf[r, pl.ds(L, L)])

            z = jnp.zeros((L,), jnp.float32)
            a0, a1 = lax.fori_loop(0, H, add_row, (z, z), unroll=5)
            row = RPD * j + h
            out_v[row, pl.ds(0, L)] = a0 * INV_H
            out_v[row, pl.ds(L, L)] = a1 * INV_H

    start(0, buf0, sem0)
    start(1, buf1, sem1)

    def ring(i, _):
        j0 = 2 * i
        wait(j0, buf0, sem0)
        reduce_pair(j0, buf0)
        start(j0 + 2, buf0, sem0)
        wait(j0 + 1, buf1, sem1)
        reduce_pair(j0 + 1, buf1)
        start(j0 + 3, buf1, sem1)
        return 0

    lax.fori_loop(0, NDMA // 2 - 1, ring, 0)

    wait(NDMA - 2, buf0, sem0)
    reduce_pair(NDMA - 2, buf0)
    wait(NDMA - 1, buf1, sem1)
    reduce_pair(NDMA - 1, buf1)

    pltpu.sync_copy(out_v, out_hbm.at[pl.ds(wid * BPW, BPW)])


def kernel(input_ids, pretrained_embeddings):
    ids = input_ids.reshape(NW, NDMA, IPD).astype(jnp.int32)
    mesh = plsc.VectorSubcoreMesh(
        core_axis_name="c", subcore_axis_name="s",
        num_cores=NC, num_subcores=NS,
    )
    run = pl.kernel(
        _body,
        out_type=jax.ShapeDtypeStruct((B, D), jnp.float32),
        mesh=mesh,
        compiler_params=pltpu.CompilerParams(use_tc_tiling_on_sc=False),
        scratch_types=[
            pltpu.VMEM((NDMA, IPD), jnp.int32),
            pltpu.VMEM((IPD, D), jnp.float32),
            pltpu.VMEM((IPD, D), jnp.float32),
            pltpu.VMEM((BPW, D), jnp.float32),
            pltpu.SemaphoreType.DMA,
            pltpu.SemaphoreType.DMA,
        ],
    )
    return run(ids, pretrained_embeddings)
